# baseline (device time: 84909 ns/iter reference)
import jax
import jax.numpy as jnp
from jax import lax
from jax.experimental import pallas as pl
from jax.experimental.pallas import tpu as pltpu

N_DEV = 4
SQ, D = 512, 1024
HL, DH, SKV = 8, 128, 2048
SCALE = 0.08838834764831843


def kernel(x, Wq, Wo, K_ext, V_ext):
    bf = jnp.bfloat16
    xb = x.reshape(SQ, D).astype(bf)
    wqb = Wq.astype(bf)
    wob = Wo.astype(bf)
    kb = K_ext.reshape(SKV, HL * DH).astype(bf)
    vb = V_ext.reshape(SKV, HL * DH).astype(bf)

    def body(x_ref, wq_ref, wo_ref, k_ref, v_ref, out_ref,
             comm_ref, send_sems, recv_sems):
        my = lax.axis_index("i")
        left = lax.rem(my + N_DEV - 1, N_DEV)
        right = lax.rem(my + 1, N_DEV)

        q = lax.dot_general(
            x_ref[:, :], wq_ref[:, :], (((1,), (0,)), ((), ())),
            preferred_element_type=jnp.float32,
        ).astype(jnp.bfloat16)
        kv = k_ref[:, :]
        vv = v_ref[:, :]
        outs = []
        for h in range(HL):
            sl = slice(h * DH, (h + 1) * DH)
            s = lax.dot_general(
                q[:, sl], kv[:, sl], (((1,), (1,)), ((), ())),
                preferred_element_type=jnp.float32,
            ) * SCALE
            m = jnp.max(s, axis=1, keepdims=True)
            p = jnp.exp(s - m)
            l = jnp.sum(p, axis=1, keepdims=True)
            o = lax.dot_general(
                p.astype(jnp.bfloat16), vv[:, sl], (((1,), (0,)), ((), ())),
                preferred_element_type=jnp.float32,
            )
            outs.append((o / l).astype(jnp.bfloat16))
        ao = jnp.concatenate(outs, axis=1)
        partial = lax.dot_general(
            ao, wo_ref[:, :], (((1,), (0,)), ((), ())),
            preferred_element_type=jnp.float32,
        )

        comm_ref[0, :, :] = partial.astype(jnp.bfloat16)

        barrier_sem = pltpu.get_barrier_semaphore()
        for nbr in (left, right):
            pl.semaphore_signal(
                barrier_sem, inc=1,
                device_id=(nbr,), device_id_type=pl.DeviceIdType.MESH,
            )
        pl.semaphore_wait(barrier_sem, 2)

        acc = partial
        for h in range(N_DEV - 1):
            rdma = pltpu.make_async_remote_copy(
                src_ref=comm_ref.at[h],
                dst_ref=comm_ref.at[h + 1],
                send_sem=send_sems.at[h],
                recv_sem=recv_sems.at[h],
                device_id=(right,),
                device_id_type=pl.DeviceIdType.MESH,
            )
            rdma.start()
            rdma.wait()
            acc = acc + comm_ref[h + 1, :, :].astype(jnp.float32)
        out_ref[:, :] = acc

    out = pl.pallas_call(
        body,
        out_shape=jax.ShapeDtypeStruct((SQ, D), jnp.float32),
        in_specs=[pl.BlockSpec(memory_space=pltpu.VMEM)] * 5,
        out_specs=pl.BlockSpec(memory_space=pltpu.VMEM),
        scratch_shapes=[
            pltpu.VMEM((N_DEV, SQ, D), jnp.bfloat16),
            pltpu.SemaphoreType.DMA((N_DEV - 1,)),
            pltpu.SemaphoreType.DMA((N_DEV - 1,)),
        ],
        compiler_params=pltpu.CompilerParams(
            collective_id=0,
            vmem_limit_bytes=56 * 1024 * 1024,
        ),
    )(xb, wqb, wob, kb, vb)
    return out.reshape(1, SQ, D)


# device time: 68670 ns/iter; 1.2365x vs baseline; 1.2365x over previous
import jax
import jax.numpy as jnp
from jax import lax
from jax.experimental import pallas as pl
from jax.experimental.pallas import tpu as pltpu

N_DEV = 4
SQ, D = 512, 1024
HSQ = SQ // 2
HL, DH, SKV = 8, 128, 2048
SCALE = 0.08838834764831843


def kernel(x, Wq, Wo, K_ext, V_ext):
    xb = x.reshape(SQ, D)
    kb = K_ext.reshape(SKV, HL * DH)
    vb = V_ext.reshape(SKV, HL * DH)

    def body(x_ref, wq_ref, wo_ref, k_ref, v_ref, out_ref,
             comm_r, comm_l, send_r, recv_r, send_l, recv_l):
        my = lax.axis_index("i")
        left = lax.rem(my + N_DEV - 1, N_DEV)
        right = lax.rem(my + 1, N_DEV)
        bf = jnp.bfloat16

        xv = x_ref[:, :].astype(bf)
        wq = wq_ref[:, :].astype(bf)
        q = lax.dot_general(
            xv, wq, (((1,), (0,)), ((), ())),
            preferred_element_type=jnp.float32,
        ).astype(bf)
        kv = k_ref[:, :].astype(bf)
        vv = v_ref[:, :].astype(bf)

        outs = []
        for h in range(HL):
            sl = slice(h * DH, (h + 1) * DH)
            s = lax.dot_general(
                q[:, sl], kv[:, sl], (((1,), (1,)), ((), ())),
                preferred_element_type=jnp.float32,
            ) * SCALE
            m = jnp.max(s, axis=1, keepdims=True)
            p = jnp.exp(s - m)
            l = jnp.sum(p, axis=1, keepdims=True)
            o = lax.dot_general(
                p.astype(bf), vv[:, sl], (((1,), (0,)), ((), ())),
                preferred_element_type=jnp.float32,
            )
            outs.append((o / l).astype(bf))
        ao = jnp.concatenate(outs, axis=1)
        wo = wo_ref[:, :].astype(bf)
        partial = lax.dot_general(
            ao, wo, (((1,), (0,)), ((), ())),
            preferred_element_type=jnp.float32,
        )

        comm_r[0, :, :] = partial[0:HSQ, :].astype(bf)
        comm_l[0, :, :] = partial[HSQ:SQ, :].astype(bf)

        barrier_sem = pltpu.get_barrier_semaphore()
        for nbr in (left, right):
            pl.semaphore_signal(
                barrier_sem, inc=1,
                device_id=(nbr,), device_id_type=pl.DeviceIdType.MESH,
            )
        pl.semaphore_wait(barrier_sem, 2)

        def hop(hp, comm, send_sems, recv_sems, dst):
            return pltpu.make_async_remote_copy(
                src_ref=comm.at[hp],
                dst_ref=comm.at[hp + 1],
                send_sem=send_sems.at[hp],
                recv_sem=recv_sems.at[hp],
                device_id=(dst,),
                device_id_type=pl.DeviceIdType.MESH,
            )

        hops_r = [hop(i, comm_r, send_r, recv_r, right)
                  for i in range(N_DEV - 1)]
        hops_l = [hop(i, comm_l, send_l, recv_l, left)
                  for i in range(N_DEV - 1)]

        hops_r[0].start()
        hops_l[0].start()
        acc_r = partial[0:HSQ, :]
        acc_l = partial[HSQ:SQ, :]
        for hp in range(1, N_DEV - 1):
            hops_r[hp - 1].wait_recv()
            hops_r[hp].start()
            hops_l[hp - 1].wait_recv()
            hops_l[hp].start()
            acc_r = acc_r + comm_r[hp, :, :].astype(jnp.float32)
            acc_l = acc_l + comm_l[hp, :, :].astype(jnp.float32)
        hops_r[N_DEV - 2].wait_recv()
        hops_l[N_DEV - 2].wait_recv()
        acc_r = acc_r + comm_r[N_DEV - 1, :, :].astype(jnp.float32)
        acc_l = acc_l + comm_l[N_DEV - 1, :, :].astype(jnp.float32)
        out_ref[0:HSQ, :] = acc_r
        out_ref[HSQ:SQ, :] = acc_l
        for hp in range(N_DEV - 1):
            hops_r[hp].wait_send()
            hops_l[hp].wait_send()

    out = pl.pallas_call(
        body,
        out_shape=jax.ShapeDtypeStruct((SQ, D), jnp.float32),
        in_specs=[pl.BlockSpec(memory_space=pltpu.VMEM)] * 5,
        out_specs=pl.BlockSpec(memory_space=pltpu.VMEM),
        scratch_shapes=[
            pltpu.VMEM((N_DEV, HSQ, D), jnp.bfloat16),
            pltpu.VMEM((N_DEV, HSQ, D), jnp.bfloat16),
            pltpu.SemaphoreType.DMA((N_DEV - 1,)),
            pltpu.SemaphoreType.DMA((N_DEV - 1,)),
            pltpu.SemaphoreType.DMA((N_DEV - 1,)),
            pltpu.SemaphoreType.DMA((N_DEV - 1,)),
        ],
        compiler_params=pltpu.CompilerParams(
            collective_id=0,
            vmem_limit_bytes=56 * 1024 * 1024,
        ),
    )(xb, Wq, Wo, kb, vb)
    return out.reshape(1, SQ, D)
